# baseline (speedup 1.0000x reference)
import functools

import jax
import jax.numpy as jnp
from jax import lax
from jax.experimental import pallas as pl
from jax.experimental.pallas import tpu as pltpu
from jax.experimental.pallas import tpu_sc as plsc

_B = 1024
_D = 64
_L = 16

_INFO = plsc.get_sparse_core_info()
_NC = _INFO.num_cores
_NS = _INFO.num_subcores
_NW = _NC * _NS
_BPW = _B // _NW


def _body(i1_hbm, i2_hbm, w2_hbm, out2_hbm,
          idx1_v, idx2_v, r1_v, r2_v, rows1_v, rows2_v, out_v, sem1, sem2):
    wid = lax.axis_index("s") * _NC + lax.axis_index("c")
    base = wid * _BPW

    pltpu.sync_copy(i1_hbm.at[pl.ds(base, _BPW)], idx1_v)
    pltpu.sync_copy(i2_hbm.at[pl.ds(base, _BPW)], idx2_v)

    for c in range(_BPW // _L):
        sl = pl.ds(c * _L, _L)
        r1_v[sl] = idx1_v[sl] >> 1
        r2_v[sl] = idx2_v[sl] >> 1

    c1 = pltpu.async_copy(w2_hbm.at[r1_v], rows1_v, sem1)
    c2 = pltpu.async_copy(w2_hbm.at[r2_v], rows2_v, sem2)
    c1.wait()
    c2.wait()

    half = jnp.full((_L,), 0.5, jnp.float32)
    one = jnp.full((_L,), 1.0, jnp.float32)
    for c in range(_BPW // _L):
        chunk1 = idx1_v[pl.ds(c * _L, _L)]
        chunk2 = idx2_v[pl.ds(c * _L, _L)]
        for j in range(_L):
            i = c * _L + j
            bidx = jnp.full((_L,), j, jnp.int32)
            v1 = jnp.take_along_axis(chunk1, bidx, axis=0,
                                     mode="promise_in_bounds")
            v2 = jnp.take_along_axis(chunk2, bidx, axis=0,
                                     mode="promise_in_bounds")
            s = jnp.where(v1 == v2, half, one)
            m1 = (v1 & 1) == 1
            m2 = (v2 & 1) == 1
            for d in range(_D // _L):
                lo = pl.ds(d * _L, _L)
                hi = pl.ds(_D + d * _L, _L)
                a = jnp.where(m1, rows1_v[i, hi], rows1_v[i, lo])
                b = jnp.where(m2, rows2_v[i, hi], rows2_v[i, lo])
                out_v[i // 2, pl.ds((i % 2) * _D + d * _L, _L)] = (a + b) * s

    pltpu.sync_copy(out_v, out2_hbm.at[pl.ds(wid * (_BPW // 2), _BPW // 2)])


@functools.partial(
    pl.kernel,
    mesh=plsc.VectorSubcoreMesh(core_axis_name="c", subcore_axis_name="s"),
    out_type=jax.ShapeDtypeStruct((_B // 2, 2 * _D), jnp.float32),
    scratch_types=[
        pltpu.VMEM((_BPW,), jnp.int32),
        pltpu.VMEM((_BPW,), jnp.int32),
        pltpu.VMEM((_BPW,), jnp.int32),
        pltpu.VMEM((_BPW,), jnp.int32),
        pltpu.VMEM((_BPW, 2 * _D), jnp.float32),
        pltpu.VMEM((_BPW, 2 * _D), jnp.float32),
        pltpu.VMEM((_BPW // 2, 2 * _D), jnp.float32),
        pltpu.SemaphoreType.DMA,
        pltpu.SemaphoreType.DMA,
    ],
    compiler_params=pltpu.CompilerParams(needs_layout_passes=False),
)
def _two_hot_sc(i1_hbm, i2_hbm, w2_hbm, out2_hbm, *scratch):
    _body(i1_hbm, i2_hbm, w2_hbm, out2_hbm, *scratch)


def kernel(input_one, input_two, W):
    w2 = W.reshape(W.shape[0] // 2, 2 * W.shape[1])
    out2 = _two_hot_sc(input_one.astype(jnp.int32),
                       input_two.astype(jnp.int32), w2)
    return out2.reshape(_B, _D)


# speedup vs baseline: 1.4510x; 1.4510x over previous
import functools

import jax
import jax.numpy as jnp
from jax import lax
from jax.experimental import pallas as pl
from jax.experimental.pallas import tpu as pltpu
from jax.experimental.pallas import tpu_sc as plsc

_B = 1024
_V = 100000
_D = 64
_L = 16
_WIN = 128
_LAST = (_V // _WIN) - 1
_TAIL = (_LAST + 1) * _WIN
_TW = _V - _TAIL

_INFO = plsc.get_sparse_core_info()
_NC = _INFO.num_cores
_NS = _INFO.num_subcores
_NW = _NC * _NS
_BPW = _B // _NW


def _body(i1_hbm, i2_hbm, wt_hbm, out2_hbm,
          idx1_v, idx2_v, blk0, blk1, blk2, blk3, tail_v, out_v,
          sem0, sem1, sem2, sem3):
    wid = lax.axis_index("s") * _NC + lax.axis_index("c")
    base = wid * _BPW
    blks = (blk0, blk1, blk2, blk3)
    sems = (sem0, sem1, sem2, sem3)

    pltpu.sync_copy(i1_hbm.at[pl.ds(base, _BPW)], idx1_v)
    pltpu.sync_copy(i2_hbm.at[pl.ds(base, _BPW)], idx2_v)
    pltpu.sync_copy(wt_hbm.at[:, pl.ds(_TAIL, _TW)], tail_v)

    handles = {}

    def issue(i):
        if i >= _BPW:
            return
        c, j = divmod(i, _L)
        bidx = jnp.full((_L,), j, jnp.int32)
        chunk1 = idx1_v[pl.ds(c * _L, _L)]
        chunk2 = idx2_v[pl.ds(c * _L, _L)]
        v1 = jnp.take_along_axis(chunk1, bidx, axis=0,
                                 mode="promise_in_bounds")
        v2 = jnp.take_along_axis(chunk2, bidx, axis=0,
                                 mode="promise_in_bounds")
        w1 = jnp.max(jnp.minimum(v1 >> 7, _LAST)) * _WIN
        w2 = jnp.max(jnp.minimum(v2 >> 7, _LAST)) * _WIN
        b0, b1 = blks[2 * (i % 2)], blks[2 * (i % 2) + 1]
        s0, s1 = sems[2 * (i % 2)], sems[2 * (i % 2) + 1]
        h1 = pltpu.async_copy(wt_hbm.at[:, pl.ds(w1, _WIN)], b0, s0)
        h2 = pltpu.async_copy(wt_hbm.at[:, pl.ds(w2, _WIN)], b1, s1)
        handles[i] = (h1, h2, v1, v2)

    issue(0)
    issue(1)

    half = jnp.full((_L,), 0.5, jnp.float32)
    one = jnp.full((_L,), 1.0, jnp.float32)
    lanes = lax.iota(jnp.int32, _L)

    def pick(blk, v, e_vec):
        lane = v - jnp.minimum(v >> 7, _LAST) * _WIN
        in_win = lane < _WIN
        main = plsc.load_gather(blk, [e_vec, jnp.minimum(lane, _WIN - 1)])
        tl = plsc.load_gather(
            tail_v,
            [e_vec, jnp.clip(lane - _WIN, 0, _TW - 1)])
        return jnp.where(in_win, main, tl)

    for i in range(_BPW):
        h1, h2, v1, v2 = handles.pop(i)
        h1.wait()
        h2.wait()
        s = jnp.where(v1 == v2, half, one)
        b0, b1 = blks[2 * (i % 2)], blks[2 * (i % 2) + 1]
        for d in range(_D // _L):
            e_vec = lanes + d * _L
            a = pick(b0, v1, e_vec)
            b = pick(b1, v2, e_vec)
            out_v[i // 2, pl.ds((i % 2) * _D + d * _L, _L)] = (a + b) * s
        issue(i + 2)

    pltpu.sync_copy(out_v, out2_hbm.at[pl.ds(wid * (_BPW // 2), _BPW // 2)])


@functools.partial(
    pl.kernel,
    mesh=plsc.VectorSubcoreMesh(core_axis_name="c", subcore_axis_name="s"),
    out_type=jax.ShapeDtypeStruct((_B // 2, 2 * _D), jnp.float32),
    scratch_types=[
        pltpu.VMEM((_BPW,), jnp.int32),
        pltpu.VMEM((_BPW,), jnp.int32),
        pltpu.VMEM((_D, _WIN), jnp.float32),
        pltpu.VMEM((_D, _WIN), jnp.float32),
        pltpu.VMEM((_D, _WIN), jnp.float32),
        pltpu.VMEM((_D, _WIN), jnp.float32),
        pltpu.VMEM((_D, _TW), jnp.float32),
        pltpu.VMEM((_BPW // 2, 2 * _D), jnp.float32),
        pltpu.SemaphoreType.DMA,
        pltpu.SemaphoreType.DMA,
        pltpu.SemaphoreType.DMA,
        pltpu.SemaphoreType.DMA,
    ],
    compiler_params=pltpu.CompilerParams(needs_layout_passes=False),
)
def _two_hot_sc(i1_hbm, i2_hbm, wt_hbm, out2_hbm, *scratch):
    _body(i1_hbm, i2_hbm, wt_hbm, out2_hbm, *scratch)


def kernel(input_one, input_two, W):
    out2 = _two_hot_sc(input_one.astype(jnp.int32),
                       input_two.astype(jnp.int32), W.T)
    return out2.reshape(_B, _D)


# speedup vs baseline: 1.6132x; 1.1117x over previous
import functools

import jax
import jax.numpy as jnp
from jax import lax
from jax.experimental import pallas as pl
from jax.experimental.pallas import tpu as pltpu
from jax.experimental.pallas import tpu_sc as plsc

_B = 1024
_V = 100000
_D = 64
_L = 16
_WIN = 128
_LAST = (_V // _WIN) - 1
_TAIL = (_LAST + 1) * _WIN
_TW = _V - _TAIL

_INFO = plsc.get_sparse_core_info()
_NC = _INFO.num_cores
_NS = _INFO.num_subcores
_NW = _NC * _NS
_BPW = _B // _NW


def _body(i1_hbm, i2_hbm, wt_hbm, out2_hbm,
          idx1_v, idx2_v, blk0, blk1, blk2, blk3, blk4, blk5, blk6, blk7,
          tail_v, out_v, sem0, sem1, sem2, sem3, sem4, sem5, sem6, sem7):
    wid = lax.axis_index("s") * _NC + lax.axis_index("c")
    base = wid * _BPW
    blks = (blk0, blk1, blk2, blk3, blk4, blk5, blk6, blk7)
    sems = (sem0, sem1, sem2, sem3, sem4, sem5, sem6, sem7)

    pltpu.sync_copy(i1_hbm.at[pl.ds(base, _BPW)], idx1_v)
    pltpu.sync_copy(i2_hbm.at[pl.ds(base, _BPW)], idx2_v)
    pltpu.sync_copy(wt_hbm.at[:, pl.ds(_TAIL, _TW)], tail_v)

    handles = {}

    def issue(i):
        if i >= _BPW:
            return
        c, j = divmod(i, _L)
        bidx = jnp.full((_L,), j, jnp.int32)
        chunk1 = idx1_v[pl.ds(c * _L, _L)]
        chunk2 = idx2_v[pl.ds(c * _L, _L)]
        v1 = jnp.take_along_axis(chunk1, bidx, axis=0,
                                 mode="promise_in_bounds")
        v2 = jnp.take_along_axis(chunk2, bidx, axis=0,
                                 mode="promise_in_bounds")
        w1 = jnp.max(jnp.minimum(v1 >> 7, _LAST)) * _WIN
        w2 = jnp.max(jnp.minimum(v2 >> 7, _LAST)) * _WIN
        b0, b1 = blks[2 * (i % 4)], blks[2 * (i % 4) + 1]
        s0, s1 = sems[2 * (i % 4)], sems[2 * (i % 4) + 1]
        h1 = pltpu.async_copy(wt_hbm.at[:, pl.ds(w1, _WIN)], b0, s0)
        h2 = pltpu.async_copy(wt_hbm.at[:, pl.ds(w2, _WIN)], b1, s1)
        handles[i] = (h1, h2, v1, v2)

    for _p in range(4):
        issue(_p)

    half = jnp.full((_L,), 0.5, jnp.float32)
    one = jnp.full((_L,), 1.0, jnp.float32)
    lanes = lax.iota(jnp.int32, _L)

    def pick(blk, v, e_vec):
        lane = v - jnp.minimum(v >> 7, _LAST) * _WIN
        in_win = lane < _WIN
        main = plsc.load_gather(blk, [e_vec, jnp.minimum(lane, _WIN - 1)])
        tl = plsc.load_gather(
            tail_v,
            [e_vec, jnp.clip(lane - _WIN, 0, _TW - 1)])
        return jnp.where(in_win, main, tl)

    for i in range(_BPW):
        h1, h2, v1, v2 = handles.pop(i)
        h1.wait()
        h2.wait()
        s = jnp.where(v1 == v2, half, one)
        b0, b1 = blks[2 * (i % 4)], blks[2 * (i % 4) + 1]
        for d in range(_D // _L):
            e_vec = lanes + d * _L
            a = pick(b0, v1, e_vec)
            b = pick(b1, v2, e_vec)
            out_v[i // 2, pl.ds((i % 2) * _D + d * _L, _L)] = (a + b) * s
        issue(i + 4)

    pltpu.sync_copy(out_v, out2_hbm.at[pl.ds(wid * (_BPW // 2), _BPW // 2)])


@functools.partial(
    pl.kernel,
    mesh=plsc.VectorSubcoreMesh(core_axis_name="c", subcore_axis_name="s"),
    out_type=jax.ShapeDtypeStruct((_B // 2, 2 * _D), jnp.float32),
    scratch_types=[
        pltpu.VMEM((_BPW,), jnp.int32),
        pltpu.VMEM((_BPW,), jnp.int32),
        pltpu.VMEM((_D, _WIN), jnp.float32),
        pltpu.VMEM((_D, _WIN), jnp.float32),
        pltpu.VMEM((_D, _WIN), jnp.float32),
        pltpu.VMEM((_D, _WIN), jnp.float32),
        pltpu.VMEM((_D, _WIN), jnp.float32),
        pltpu.VMEM((_D, _WIN), jnp.float32),
        pltpu.VMEM((_D, _WIN), jnp.float32),
        pltpu.VMEM((_D, _WIN), jnp.float32),
        pltpu.VMEM((_D, _TW), jnp.float32),
        pltpu.VMEM((_BPW // 2, 2 * _D), jnp.float32),
        pltpu.SemaphoreType.DMA,
        pltpu.SemaphoreType.DMA,
        pltpu.SemaphoreType.DMA,
        pltpu.SemaphoreType.DMA,
        pltpu.SemaphoreType.DMA,
        pltpu.SemaphoreType.DMA,
        pltpu.SemaphoreType.DMA,
        pltpu.SemaphoreType.DMA,
    ],
    compiler_params=pltpu.CompilerParams(needs_layout_passes=False),
)
def _two_hot_sc(i1_hbm, i2_hbm, wt_hbm, out2_hbm, *scratch):
    _body(i1_hbm, i2_hbm, wt_hbm, out2_hbm, *scratch)


def kernel(input_one, input_two, W):
    out2 = _two_hot_sc(input_one.astype(jnp.int32),
                       input_two.astype(jnp.int32), W.T)
    return out2.reshape(_B, _D)


# speedup vs baseline: 1.6165x; 1.0021x over previous
import functools

import jax
import jax.numpy as jnp
from jax import lax
from jax.experimental import pallas as pl
from jax.experimental.pallas import tpu as pltpu
from jax.experimental.pallas import tpu_sc as plsc

_B = 1024
_V = 100000
_D = 64
_L = 16
_WIN = 128
_LAST = (_V // _WIN) - 1
_TAIL = (_LAST + 1) * _WIN
_TW = _V - _TAIL

_INFO = plsc.get_sparse_core_info()
_NC = _INFO.num_cores
_NS = _INFO.num_subcores
_NW = _NC * _NS
_BPW = _B // _NW


def _body(i1_hbm, i2_hbm, wt_hbm, out2_hbm,
          idx1_v, idx2_v, blk0, blk1, blk2, blk3, blk4, blk5, blk6, blk7, blk8, blk9, blk10, blk11,
          tail_v, out_v, sem0, sem1, sem2, sem3, sem4, sem5, sem6, sem7,
          sem8, sem9, sem10, sem11):
    wid = lax.axis_index("s") * _NC + lax.axis_index("c")
    base = wid * _BPW
    blks = (blk0, blk1, blk2, blk3, blk4, blk5, blk6, blk7, blk8, blk9, blk10, blk11)
    sems = (sem0, sem1, sem2, sem3, sem4, sem5, sem6, sem7, sem8, sem9, sem10, sem11)

    pltpu.sync_copy(i1_hbm.at[pl.ds(base, _BPW)], idx1_v)
    pltpu.sync_copy(i2_hbm.at[pl.ds(base, _BPW)], idx2_v)
    pltpu.sync_copy(wt_hbm.at[:, pl.ds(_TAIL, _TW)], tail_v)

    handles = {}

    def issue(i):
        if i >= _BPW:
            return
        c, j = divmod(i, _L)
        bidx = jnp.full((_L,), j, jnp.int32)
        chunk1 = idx1_v[pl.ds(c * _L, _L)]
        chunk2 = idx2_v[pl.ds(c * _L, _L)]
        v1 = jnp.take_along_axis(chunk1, bidx, axis=0,
                                 mode="promise_in_bounds")
        v2 = jnp.take_along_axis(chunk2, bidx, axis=0,
                                 mode="promise_in_bounds")
        w1 = jnp.max(jnp.minimum(v1 >> 7, _LAST)) * _WIN
        w2 = jnp.max(jnp.minimum(v2 >> 7, _LAST)) * _WIN
        b0, b1 = blks[2 * (i % 6)], blks[2 * (i % 6) + 1]
        s0, s1 = sems[2 * (i % 6)], sems[2 * (i % 6) + 1]
        h1 = pltpu.async_copy(wt_hbm.at[:, pl.ds(w1, _WIN)], b0, s0)
        h2 = pltpu.async_copy(wt_hbm.at[:, pl.ds(w2, _WIN)], b1, s1)
        handles[i] = (h1, h2, v1, v2)

    for _p in range(6):
        issue(_p)

    half = jnp.full((_L,), 0.5, jnp.float32)
    one = jnp.full((_L,), 1.0, jnp.float32)
    lanes = lax.iota(jnp.int32, _L)

    def pick(blk, v, e_vec):
        lane = v - jnp.minimum(v >> 7, _LAST) * _WIN
        in_win = lane < _WIN
        main = plsc.load_gather(blk, [e_vec, jnp.minimum(lane, _WIN - 1)])
        tl = plsc.load_gather(
            tail_v,
            [e_vec, jnp.clip(lane - _WIN, 0, _TW - 1)])
        return jnp.where(in_win, main, tl)

    for i in range(_BPW):
        h1, h2, v1, v2 = handles.pop(i)
        h1.wait()
        h2.wait()
        s = jnp.where(v1 == v2, half, one)
        b0, b1 = blks[2 * (i % 6)], blks[2 * (i % 6) + 1]
        for d in range(_D // _L):
            e_vec = lanes + d * _L
            a = pick(b0, v1, e_vec)
            b = pick(b1, v2, e_vec)
            out_v[i // 2, pl.ds((i % 2) * _D + d * _L, _L)] = (a + b) * s
        issue(i + 6)

    pltpu.sync_copy(out_v, out2_hbm.at[pl.ds(wid * (_BPW // 2), _BPW // 2)])


@functools.partial(
    pl.kernel,
    mesh=plsc.VectorSubcoreMesh(core_axis_name="c", subcore_axis_name="s"),
    out_type=jax.ShapeDtypeStruct((_B // 2, 2 * _D), jnp.float32),
    scratch_types=[
        pltpu.VMEM((_BPW,), jnp.int32),
        pltpu.VMEM((_BPW,), jnp.int32),
        pltpu.VMEM((_D, _WIN), jnp.float32),
        pltpu.VMEM((_D, _WIN), jnp.float32),
        pltpu.VMEM((_D, _WIN), jnp.float32),
        pltpu.VMEM((_D, _WIN), jnp.float32),
        pltpu.VMEM((_D, _WIN), jnp.float32),
        pltpu.VMEM((_D, _WIN), jnp.float32),
        pltpu.VMEM((_D, _WIN), jnp.float32),
        pltpu.VMEM((_D, _WIN), jnp.float32),
        pltpu.VMEM((_D, _WIN), jnp.float32),
        pltpu.VMEM((_D, _WIN), jnp.float32),
        pltpu.VMEM((_D, _WIN), jnp.float32),
        pltpu.VMEM((_D, _WIN), jnp.float32),
        pltpu.VMEM((_D, _TW), jnp.float32),
        pltpu.VMEM((_BPW // 2, 2 * _D), jnp.float32),
        pltpu.SemaphoreType.DMA,
        pltpu.SemaphoreType.DMA,
        pltpu.SemaphoreType.DMA,
        pltpu.SemaphoreType.DMA,
        pltpu.SemaphoreType.DMA,
        pltpu.SemaphoreType.DMA,
        pltpu.SemaphoreType.DMA,
        pltpu.SemaphoreType.DMA,
        pltpu.SemaphoreType.DMA,
        pltpu.SemaphoreType.DMA,
        pltpu.SemaphoreType.DMA,
        pltpu.SemaphoreType.DMA,
    ],
    compiler_params=pltpu.CompilerParams(needs_layout_passes=False),
)
def _two_hot_sc(i1_hbm, i2_hbm, wt_hbm, out2_hbm, *scratch):
    _body(i1_hbm, i2_hbm, wt_hbm, out2_hbm, *scratch)


def kernel(input_one, input_two, W):
    out2 = _two_hot_sc(input_one.astype(jnp.int32),
                       input_two.astype(jnp.int32), W.T)
    return out2.reshape(_B, _D)
